# Initial kernel scaffold; baseline (speedup 1.0000x reference)
import jax
import jax.numpy as jnp
from jax.experimental import pallas as pl


def kernel(energy, batch, W1, b1, W2, b2):
    raise NotImplementedError("write your pallas kernel here")


# speedup vs baseline: 2.5990x; 2.5990x over previous
import jax
import jax.numpy as jnp
from jax.experimental import pallas as pl

D_MODEL = 256
N_NODES = 50000
N_MOL = 512
BLOCK = 2000
NB = N_NODES // BLOCK


def _fused_kernel(batch_ref, energy_ref, w1_ref, b1_ref, w2_ref, b2_ref, out_ref):
    i = pl.program_id(0)
    h = jnp.dot(energy_ref[...], w1_ref[...], preferred_element_type=jnp.float32)
    h = h + b1_ref[...]
    h = h * jax.nn.sigmoid(h)
    out = jnp.dot(h, w2_ref[...], preferred_element_type=jnp.float32) + b2_ref[...]
    ids = batch_ref[0, 0, :]
    mol = jax.lax.broadcasted_iota(jnp.int32, (BLOCK, N_MOL), 1)
    contrib = jnp.where(ids[:, None] == mol, out, 0.0)
    partial = jnp.sum(contrib, axis=0)[None, :]

    @pl.when(i == 0)
    def _init():
        out_ref[...] = jnp.zeros_like(out_ref)

    out_ref[...] += partial


def kernel(energy, batch, W1, b1, W2, b2):
    batch3 = batch.astype(jnp.int32).reshape(NB, 1, BLOCK)
    b1r = b1.reshape(1, D_MODEL)
    b2r = b2.reshape(1, 1)
    pooled = pl.pallas_call(
        _fused_kernel,
        grid=(NB,),
        in_specs=[
            pl.BlockSpec((1, 1, BLOCK), lambda i: (i, 0, 0)),
            pl.BlockSpec((BLOCK, D_MODEL), lambda i: (i, 0)),
            pl.BlockSpec((D_MODEL, D_MODEL), lambda i: (0, 0)),
            pl.BlockSpec((1, D_MODEL), lambda i: (0, 0)),
            pl.BlockSpec((D_MODEL, 1), lambda i: (0, 0)),
            pl.BlockSpec((1, 1), lambda i: (0, 0)),
        ],
        out_specs=pl.BlockSpec((1, N_MOL), lambda i: (0, 0)),
        out_shape=jax.ShapeDtypeStruct((1, N_MOL), jnp.float32),
    )(batch3, energy, W1, b1r, W2, b2r)
    return pooled[0]
